# baseline (speedup 1.0000x reference)
import functools

import jax
import jax.numpy as jnp
from jax import lax
from jax.experimental import pallas as pl
from jax.experimental.pallas import tpu as pltpu
from jax.experimental.pallas import tpu_sc as plsc

N = 100000
E = 3200000
G = 1024

NC = 2
NS = 16
HALF = N // NC
GSPREAD = 512
ROWS = HALF + GSPREAD
XP = 24
AE = 8

ZROWS = 3200
NBLK = 98
NPAD = NBLK * 1024

CHX = 512
NCHX = E // CHX
FKX = NCHX // NS
EXX = NCHX - FKX * NS

CHA = 512
NCHA = E // CHA
FKA = NCHA // NS
EXA = NCHA - FKA * NS


def _local_idx(dstv, lidx, cbase, iota16, nrows):
  for j in range(nrows):
    for t in range(8):
      d = dstv[j, pl.ds(t * 16, 16)]
      ld = d - cbase
      ok = (ld >= 0) & (ld < HALF)
      garb = HALF + ((j * 128 + t * 16 + iota16) & (GSPREAD - 1))
      lidx[j, pl.ds(t * 16, 16)] = jnp.where(ok, ld, garb)


def _zero_init(z_hbm, agg_sh, s):
  @pl.when(s < NS - 1)
  def _():
    pltpu.sync_copy(z_hbm.at[pl.ds(0, ZROWS)],
                    agg_sh.at[pl.ds(s * ZROWS, ZROWS)])

  @pl.when(s == NS - 1)
  def _():
    n = ROWS - (NS - 1) * ZROWS
    pltpu.sync_copy(z_hbm.at[pl.ds(0, n)],
                    agg_sh.at[pl.ds((NS - 1) * ZROWS, n)])


def _copy_out(agg_sh, agg_out, s, cbase):
  @pl.when(s < NS - 1)
  def _():
    pltpu.sync_copy(agg_sh.at[pl.ds(s * ZROWS, ZROWS)],
                    agg_out.at[pl.ds(cbase + s * ZROWS, ZROWS)])

  @pl.when(s == NS - 1)
  def _():
    n = HALF - (NS - 1) * ZROWS
    pltpu.sync_copy(agg_sh.at[pl.ds((NS - 1) * ZROWS, n)],
                    agg_out.at[pl.ds(cbase + (NS - 1) * ZROWS, n)])


def _sc_x_body(src2d, dst2d, xpad_hbm, zx_hbm, aggx_out,
               aggx_sh, srcv, dstv, lidx, xrows,
               gsem0, gsem1, ssem0, ssem1, csem):
  c = lax.axis_index("c")
  s = lax.axis_index("s")
  cbase = c * HALF
  gsem = (gsem0, gsem1)
  ssem = (ssem0, ssem1)

  _zero_init(zx_hbm, aggx_sh, s)
  plsc.subcore_barrier()

  iota16 = lax.iota(jnp.int32, 16)
  nr = CHX // 128

  def load_idx(q, p):
    d1 = pltpu.async_copy(src2d.at[pl.ds(q * nr, nr)], srcv.at[p], csem)
    d2 = pltpu.async_copy(dst2d.at[pl.ds(q * nr, nr)], dstv.at[p], csem)
    d1.wait()
    d2.wait()
    _local_idx(dstv.at[p], lidx.at[p], cbase, iota16, nr)

  def fire_gather(p):
    for j in range(nr):
      pltpu.async_copy(xpad_hbm.at[srcv.at[p, j]],
                       xrows.at[p, pl.ds(j * 128, 128)], gsem[p])

  def fire_scatter(p):
    for j in range(nr):
      pltpu.async_copy(xrows.at[p, pl.ds(j * 128, 128)],
                       aggx_sh.at[lidx.at[p, j]], ssem[p], add=True)

  def drain(sem, p):
    pltpu.make_async_copy(xpad_hbm.at[pl.ds(0, CHX)], xrows.at[p], sem).wait()

  load_idx(s, 0)
  fire_gather(0)
  load_idx(NS + s, 1)
  fire_gather(1)
  drain(gsem[0], 0)
  fire_scatter(0)

  def loop_body(m, carry):
    for half in range(2):
      p = half
      drain(ssem[p], p)
      load_idx((2 * m + half) * NS + s, p)
      fire_gather(p)
      drain(gsem[1 - p], 1 - p)
      fire_scatter(1 - p)
    return carry

  lax.fori_loop(1, FKX // 2, loop_body, 0)

  drain(gsem[1], 1)
  fire_scatter(1)
  drain(ssem[0], 0)
  drain(ssem[1], 1)

  @pl.when(s < EXX)
  def _():
    load_idx(FKX * NS + s, 0)
    fire_gather(0)
    drain(gsem[0], 0)
    fire_scatter(0)
    drain(ssem[0], 0)

  plsc.subcore_barrier()
  _copy_out(aggx_sh, aggx_out, s, cbase)


def _sc_attr_body(dst2d, attr8, za_hbm, agge_out,
                  agge_sh, dstv, lidx, attrv, ssem0, ssem1, csem):
  c = lax.axis_index("c")
  s = lax.axis_index("s")
  cbase = c * HALF
  ssem = (ssem0, ssem1)

  _zero_init(za_hbm, agge_sh, s)
  plsc.subcore_barrier()

  iota16 = lax.iota(jnp.int32, 16)
  nr = CHA // 128

  def load_chunk(q, p):
    d2 = pltpu.async_copy(dst2d.at[pl.ds(q * nr, nr)], dstv.at[p], csem)
    d3 = pltpu.async_copy(attr8.at[pl.ds(q * nr, nr)], attrv.at[p], csem)
    d2.wait()
    d3.wait()
    _local_idx(dstv.at[p], lidx.at[p], cbase, iota16, nr)

  def fire_scatter(p):
    for j in range(nr):
      pltpu.async_copy(attrv.at[p, j],
                       agge_sh.at[lidx.at[p, j]], ssem[p], add=True)

  def drain(p):
    pltpu.make_async_copy(attr8.at[pl.ds(0, nr)], attrv.at[p], ssem[p]).wait()

  load_chunk(s, 0)
  fire_scatter(0)
  load_chunk(NS + s, 1)
  fire_scatter(1)

  def loop_body(m, carry):
    for half in range(2):
      p = half
      drain(p)
      load_chunk((2 * m + half) * NS + s, p)
      fire_scatter(p)
    return carry

  lax.fori_loop(1, FKA // 2, loop_body, 0)

  drain(0)
  drain(1)

  @pl.when(s < EXA)
  def _():
    load_chunk(FKA * NS + s, 0)
    fire_scatter(0)
    drain(0)

  plsc.subcore_barrier()
  _copy_out(agge_sh, agge_out, s, cbase)


@functools.cache
def _get_sc_x():
  return pl.kernel(
      _sc_x_body,
      out_type=jax.ShapeDtypeStruct((NPAD, XP), jnp.float32),
      mesh=plsc.VectorSubcoreMesh(core_axis_name="c", subcore_axis_name="s"),
      compiler_params=pltpu.CompilerParams(use_tc_tiling_on_sc=False),
      scratch_types=[
          pltpu.VMEM_SHARED((ROWS, XP), jnp.float32),
          pltpu.VMEM((2, 4, 128), jnp.int32),
          pltpu.VMEM((2, 4, 128), jnp.int32),
          pltpu.VMEM((2, 4, 128), jnp.int32),
          pltpu.VMEM((2, CHX, XP), jnp.float32),
          pltpu.SemaphoreType.DMA,
          pltpu.SemaphoreType.DMA,
          pltpu.SemaphoreType.DMA,
          pltpu.SemaphoreType.DMA,
          pltpu.SemaphoreType.DMA,
      ],
  )


@functools.cache
def _get_sc_attr():
  return pl.kernel(
      _sc_attr_body,
      out_type=jax.ShapeDtypeStruct((NPAD, AE), jnp.float32),
      mesh=plsc.VectorSubcoreMesh(core_axis_name="c", subcore_axis_name="s"),
      compiler_params=pltpu.CompilerParams(use_tc_tiling_on_sc=False),
      scratch_types=[
          pltpu.VMEM_SHARED((ROWS, AE), jnp.float32),
          pltpu.VMEM((2, 4, 128), jnp.int32),
          pltpu.VMEM((2, 4, 128), jnp.int32),
          pltpu.VMEM((2, CHA // 128, 128, AE), jnp.float32),
          pltpu.SemaphoreType.DMA,
          pltpu.SemaphoreType.DMA,
          pltpu.SemaphoreType.DMA,
      ],
  )


def _tc_body(aggx, agge, batch3, tdf, w1a, w1b, b1r, w2, b2r,
             wflat, cbr, cfcw, cfbr, owd, owt, obr,
             out, pooled, cnt):
  i = pl.program_id(0)

  @pl.when(i == 0)
  def _():
    pooled[...] = jnp.zeros_like(pooled)
    cnt[...] = jnp.zeros_like(cnt)

  r = aggx[...] @ w1a[...] + agge[...] @ w1b[...] + b1r[...]
  r = jnp.maximum(r, 0.0)
  col = lax.broadcasted_iota(jnp.int32, (1024, 1), 0) + i * 1024
  r = jnp.where(col < N, r, 0.0)
  brow = batch3[0]
  mrow = (lax.broadcasted_iota(jnp.int32, (1, 1024), 1) + i * 1024) < N
  gi = lax.broadcasted_iota(jnp.int32, (1024, 1024), 0)
  oh = ((gi == brow) & mrow).astype(jnp.float32)
  pooled[...] += jnp.dot(oh, r, preferred_element_type=jnp.float32)
  cnt[...] += jnp.dot(oh, jnp.ones((1024, 8), jnp.float32),
                      preferred_element_type=jnp.float32)

  @pl.when(i == NBLK - 1)
  def _():
    c = jnp.maximum(cnt[:, 0:1], 1.0)
    drug = (pooled[...] / c) @ w2[...] + b2r[...]
    t0 = jnp.dot(tdf[:, 0:105], wflat[...],
                 preferred_element_type=jnp.float32)
    t1 = jnp.dot(tdf[:, 21:126], wflat[...],
                 preferred_element_type=jnp.float32)
    cc = jnp.maximum(jnp.maximum(t0, t1) + cbr[...], 0.0)
    tf = jnp.dot(cc, cfcw[...], preferred_element_type=jnp.float32) + cfbr[...]
    z = (jnp.dot(drug, owd[...], preferred_element_type=jnp.float32)
         + jnp.dot(tf, owt[...], preferred_element_type=jnp.float32)
         + obr[0, 0])
    out[...] = 1.0 / (1.0 + jnp.exp(-z))


def _tc_head(aggx, agge, batch3, tdf, w1a, w1b, b1r, w2, b2r,
             wflat, cbr, cfcw, cfbr, owd, owt, obr):
  whole = lambda *shape: pl.BlockSpec(shape, lambda i: tuple(0 for _ in shape))
  return pl.pallas_call(
      _tc_body,
      grid=(NBLK,),
      in_specs=[
          pl.BlockSpec((1024, XP), lambda i: (i, 0)),
          pl.BlockSpec((1024, AE), lambda i: (i, 0)),
          pl.BlockSpec((1, 1, 1024), lambda i: (i, 0, 0)),
          whole(1024, 126),
          whole(XP, 128),
          whole(AE, 128),
          whole(1, 128),
          whole(128, 256),
          whole(1, 256),
          whole(105, 64),
          whole(1, 64),
          whole(64, 128),
          whole(1, 128),
          whole(256, 1),
          whole(128, 1),
          whole(1, 1),
      ],
      out_specs=pl.BlockSpec((1024, 1), lambda i: (0, 0)),
      out_shape=jax.ShapeDtypeStruct((G, 1), jnp.float32),
      scratch_shapes=[
          pltpu.VMEM((1024, 128), jnp.float32),
          pltpu.VMEM((1024, 8), jnp.float32),
      ],
  )(aggx, agge, batch3, tdf, w1a, w1b, b1r, w2, b2r,
    wflat, cbr, cfcw, cfbr, owd, owt, obr)


@jax.jit
def kernel(x, edge_index, edge_attr, batch, target_data,
           W1, b1, W2, b2, conv_w, conv_b, cfc_w, cfc_b, out_w, out_b):
  src2d = edge_index[0].reshape(E // 128, 128)
  dst2d = edge_index[1].reshape(E // 128, 128)
  xpad = jnp.concatenate(
      [x, jnp.zeros((N, XP - x.shape[1]), jnp.float32)], axis=1)
  attr8 = jnp.pad(
      edge_attr.reshape(E // 128, 128, 5),
      ((0, 0), (0, 0), (0, AE - 5)))
  zx = jnp.zeros((ZROWS, XP), jnp.float32)
  za = jnp.zeros((ZROWS, AE), jnp.float32)

  aggx = _get_sc_x()(src2d, dst2d, xpad, zx)
  agge = _get_sc_attr()(dst2d, attr8, za)

  batch3 = jnp.pad(batch, (0, NPAD - N)).reshape(NBLK, 1, 1024)
  tdf = target_data.transpose(0, 2, 1).reshape(G, 126)
  w1a = jnp.concatenate(
      [W1[:20], jnp.zeros((XP - 20, 128), jnp.float32)], axis=0)
  w1b = jnp.concatenate(
      [W1[20:25], jnp.zeros((AE - 5, 128), jnp.float32)], axis=0)
  wflat = conv_w.transpose(2, 1, 0).reshape(105, 64)
  return _tc_head(
      aggx, agge, batch3, tdf, w1a, w1b, b1.reshape(1, 128),
      W2, b2.reshape(1, 256), wflat, conv_b.reshape(1, 64),
      cfc_w, cfc_b.reshape(1, 128), out_w[:256], out_w[256:],
      out_b.reshape(1, 1))


# speedup vs baseline: 1.0005x; 1.0005x over previous
import functools

import jax
import jax.numpy as jnp
from jax import lax
from jax.experimental import pallas as pl
from jax.experimental.pallas import tpu as pltpu
from jax.experimental.pallas import tpu_sc as plsc

N = 100000
E = 3200000
G = 1024

NC = 2
NS = 16
HALF = N // NC
GSPREAD = 512
ROWS = HALF + GSPREAD
XP = 24
AE = 8

ZROWS = 3200
NBLK = 98
NPAD = NBLK * 1024

CHX = 512
NCHX = E // CHX
FKX = NCHX // NS
EXX = NCHX - FKX * NS

CHA = 512
NCHA = E // CHA
FKA = NCHA // NS
EXA = NCHA - FKA * NS


def _local_idx(dstv, lidx, cbase, iota16, nrows):
  for j in range(nrows):
    for t in range(8):
      d = dstv[j, pl.ds(t * 16, 16)]
      ld = d - cbase
      ok = (ld >= 0) & (ld < HALF)
      garb = HALF + ((j * 128 + t * 16 + iota16) & (GSPREAD - 1))
      lidx[j, pl.ds(t * 16, 16)] = jnp.where(ok, ld, garb)


def _zero_init(z_hbm, agg_sh, s):
  @pl.when(s < NS - 1)
  def _():
    pltpu.sync_copy(z_hbm.at[pl.ds(0, ZROWS)],
                    agg_sh.at[pl.ds(s * ZROWS, ZROWS)])

  @pl.when(s == NS - 1)
  def _():
    n = ROWS - (NS - 1) * ZROWS
    pltpu.sync_copy(z_hbm.at[pl.ds(0, n)],
                    agg_sh.at[pl.ds((NS - 1) * ZROWS, n)])


def _copy_out(agg_sh, agg_out, s, cbase):
  @pl.when(s < NS - 1)
  def _():
    pltpu.sync_copy(agg_sh.at[pl.ds(s * ZROWS, ZROWS)],
                    agg_out.at[pl.ds(cbase + s * ZROWS, ZROWS)])

  @pl.when(s == NS - 1)
  def _():
    n = HALF - (NS - 1) * ZROWS
    pltpu.sync_copy(agg_sh.at[pl.ds((NS - 1) * ZROWS, n)],
                    agg_out.at[pl.ds(cbase + (NS - 1) * ZROWS, n)])


def _sc_x_body(src2d, dst2d, xpad_hbm, zx_hbm, aggx_out,
               aggx_sh, srcv, dstv, lidx, xrows,
               gsem0, gsem1, ssem0, ssem1, csem):
  c = lax.axis_index("c")
  s = lax.axis_index("s")
  cbase = c * HALF
  gsem = (gsem0, gsem1)
  ssem = (ssem0, ssem1)

  _zero_init(zx_hbm, aggx_sh, s)
  plsc.subcore_barrier()

  iota16 = lax.iota(jnp.int32, 16)
  nr = CHX // 128

  def load_idx(q, p):
    d1 = pltpu.async_copy(src2d.at[pl.ds(q * nr, nr)], srcv.at[p], csem)
    d2 = pltpu.async_copy(dst2d.at[pl.ds(q * nr, nr)], dstv.at[p], csem)
    d1.wait()
    d2.wait()
    _local_idx(dstv.at[p], lidx.at[p], cbase, iota16, nr)

  def fire_gather(p):
    for j in range(nr):
      pltpu.async_copy(xpad_hbm.at[srcv.at[p, j]],
                       xrows.at[p, pl.ds(j * 128, 128)], gsem[p])

  def fire_scatter(p):
    for j in range(nr):
      pltpu.async_copy(xrows.at[p, pl.ds(j * 128, 128)],
                       aggx_sh.at[lidx.at[p, j]], ssem[p], add=True)

  def drain(sem, p):
    pltpu.make_async_copy(xpad_hbm.at[pl.ds(0, CHX)], xrows.at[p], sem).wait()

  load_idx(s, 0)
  fire_gather(0)
  load_idx(NS + s, 1)
  fire_gather(1)
  drain(gsem[0], 0)
  fire_scatter(0)

  def loop_body(m, carry):
    for half in range(2):
      p = half
      drain(ssem[p], p)
      load_idx((2 * m + half) * NS + s, p)
      fire_gather(p)
      drain(gsem[1 - p], 1 - p)
      fire_scatter(1 - p)
    return carry

  lax.fori_loop(1, FKX // 2, loop_body, 0)

  drain(gsem[1], 1)
  fire_scatter(1)
  drain(ssem[0], 0)
  drain(ssem[1], 1)

  @pl.when(s < EXX)
  def _():
    load_idx(FKX * NS + s, 0)
    fire_gather(0)
    drain(gsem[0], 0)
    fire_scatter(0)
    drain(ssem[0], 0)

  plsc.subcore_barrier()
  _copy_out(aggx_sh, aggx_out, s, cbase)


def _sc_attr_body(dst2d, attr8, za_hbm, agge_out,
                  agge_sh, dstv, lidx, attrv, ssem0, ssem1, csem):
  c = lax.axis_index("c")
  s = lax.axis_index("s")
  cbase = c * HALF
  ssem = (ssem0, ssem1)

  _zero_init(za_hbm, agge_sh, s)
  plsc.subcore_barrier()

  iota16 = lax.iota(jnp.int32, 16)
  nr = CHA // 128

  def load_chunk(q, p):
    d2 = pltpu.async_copy(dst2d.at[pl.ds(q * nr, nr)], dstv.at[p], csem)
    d3 = pltpu.async_copy(attr8.at[pl.ds(q * CHA, CHA)], attrv.at[p], csem)
    d2.wait()
    d3.wait()
    _local_idx(dstv.at[p], lidx.at[p], cbase, iota16, nr)

  def fire_scatter(p):
    for j in range(nr):
      pltpu.async_copy(attrv.at[p, pl.ds(j * 128, 128)],
                       agge_sh.at[lidx.at[p, j]], ssem[p], add=True)

  def drain(p):
    pltpu.make_async_copy(attr8.at[pl.ds(0, CHA)], attrv.at[p], ssem[p]).wait()

  load_chunk(s, 0)
  fire_scatter(0)
  load_chunk(NS + s, 1)
  fire_scatter(1)

  def loop_body(m, carry):
    for half in range(2):
      p = half
      drain(p)
      load_chunk((2 * m + half) * NS + s, p)
      fire_scatter(p)
    return carry

  lax.fori_loop(1, FKA // 2, loop_body, 0)

  drain(0)
  drain(1)

  @pl.when(s < EXA)
  def _():
    load_chunk(FKA * NS + s, 0)
    fire_scatter(0)
    drain(0)

  plsc.subcore_barrier()
  _copy_out(agge_sh, agge_out, s, cbase)


@functools.cache
def _get_sc_x():
  return pl.kernel(
      _sc_x_body,
      out_type=jax.ShapeDtypeStruct((NPAD, XP), jnp.float32),
      mesh=plsc.VectorSubcoreMesh(core_axis_name="c", subcore_axis_name="s"),
      compiler_params=pltpu.CompilerParams(use_tc_tiling_on_sc=False),
      scratch_types=[
          pltpu.VMEM_SHARED((ROWS, XP), jnp.float32),
          pltpu.VMEM((2, 4, 128), jnp.int32),
          pltpu.VMEM((2, 4, 128), jnp.int32),
          pltpu.VMEM((2, 4, 128), jnp.int32),
          pltpu.VMEM((2, CHX, XP), jnp.float32),
          pltpu.SemaphoreType.DMA,
          pltpu.SemaphoreType.DMA,
          pltpu.SemaphoreType.DMA,
          pltpu.SemaphoreType.DMA,
          pltpu.SemaphoreType.DMA,
      ],
  )


@functools.cache
def _get_sc_attr():
  return pl.kernel(
      _sc_attr_body,
      out_type=jax.ShapeDtypeStruct((NPAD, AE), jnp.float32),
      mesh=plsc.VectorSubcoreMesh(core_axis_name="c", subcore_axis_name="s"),
      compiler_params=pltpu.CompilerParams(use_tc_tiling_on_sc=False),
      scratch_types=[
          pltpu.VMEM_SHARED((ROWS, AE), jnp.float32),
          pltpu.VMEM((2, 4, 128), jnp.int32),
          pltpu.VMEM((2, 4, 128), jnp.int32),
          pltpu.VMEM((2, CHA, AE), jnp.float32),
          pltpu.SemaphoreType.DMA,
          pltpu.SemaphoreType.DMA,
          pltpu.SemaphoreType.DMA,
      ],
  )


def _tc_body(aggx, agge, batch3, tdf, w1a, w1b, b1r, w2, b2r,
             wflat, cbr, cfcw, cfbr, owd, owt, obr,
             out, pooled, cnt):
  i = pl.program_id(0)

  @pl.when(i == 0)
  def _():
    pooled[...] = jnp.zeros_like(pooled)
    cnt[...] = jnp.zeros_like(cnt)

  r = aggx[...] @ w1a[...] + agge[...] @ w1b[...] + b1r[...]
  r = jnp.maximum(r, 0.0)
  col = lax.broadcasted_iota(jnp.int32, (1024, 1), 0) + i * 1024
  r = jnp.where(col < N, r, 0.0)
  brow = batch3[0]
  mrow = (lax.broadcasted_iota(jnp.int32, (1, 1024), 1) + i * 1024) < N
  gi = lax.broadcasted_iota(jnp.int32, (1024, 1024), 0)
  oh = ((gi == brow) & mrow).astype(jnp.float32)
  pooled[...] += jnp.dot(oh, r, preferred_element_type=jnp.float32)
  cnt[...] += jnp.dot(oh, jnp.ones((1024, 8), jnp.float32),
                      preferred_element_type=jnp.float32)

  @pl.when(i == NBLK - 1)
  def _():
    c = jnp.maximum(cnt[:, 0:1], 1.0)
    drug = (pooled[...] / c) @ w2[...] + b2r[...]
    t0 = jnp.dot(tdf[:, 0:105], wflat[...],
                 preferred_element_type=jnp.float32)
    t1 = jnp.dot(tdf[:, 21:126], wflat[...],
                 preferred_element_type=jnp.float32)
    cc = jnp.maximum(jnp.maximum(t0, t1) + cbr[...], 0.0)
    tf = jnp.dot(cc, cfcw[...], preferred_element_type=jnp.float32) + cfbr[...]
    z = (jnp.dot(drug, owd[...], preferred_element_type=jnp.float32)
         + jnp.dot(tf, owt[...], preferred_element_type=jnp.float32)
         + obr[0, 0])
    out[...] = 1.0 / (1.0 + jnp.exp(-z))


def _tc_head(aggx, agge, batch3, tdf, w1a, w1b, b1r, w2, b2r,
             wflat, cbr, cfcw, cfbr, owd, owt, obr):
  whole = lambda *shape: pl.BlockSpec(shape, lambda i: tuple(0 for _ in shape))
  return pl.pallas_call(
      _tc_body,
      grid=(NBLK,),
      in_specs=[
          pl.BlockSpec((1024, XP), lambda i: (i, 0)),
          pl.BlockSpec((1024, AE), lambda i: (i, 0)),
          pl.BlockSpec((1, 1, 1024), lambda i: (i, 0, 0)),
          whole(1024, 126),
          whole(XP, 128),
          whole(AE, 128),
          whole(1, 128),
          whole(128, 256),
          whole(1, 256),
          whole(105, 64),
          whole(1, 64),
          whole(64, 128),
          whole(1, 128),
          whole(256, 1),
          whole(128, 1),
          whole(1, 1),
      ],
      out_specs=pl.BlockSpec((1024, 1), lambda i: (0, 0)),
      out_shape=jax.ShapeDtypeStruct((G, 1), jnp.float32),
      scratch_shapes=[
          pltpu.VMEM((1024, 128), jnp.float32),
          pltpu.VMEM((1024, 8), jnp.float32),
      ],
  )(aggx, agge, batch3, tdf, w1a, w1b, b1r, w2, b2r,
    wflat, cbr, cfcw, cfbr, owd, owt, obr)


@jax.jit
def kernel(x, edge_index, edge_attr, batch, target_data,
           W1, b1, W2, b2, conv_w, conv_b, cfc_w, cfc_b, out_w, out_b):
  src2d = edge_index[0].reshape(E // 128, 128)
  dst2d = edge_index[1].reshape(E // 128, 128)
  xpad = jnp.concatenate(
      [x, jnp.zeros((N, XP - x.shape[1]), jnp.float32)], axis=1)
  attr8 = jnp.concatenate(
      [edge_attr, jnp.zeros((E, AE - edge_attr.shape[1]), jnp.float32)],
      axis=1)
  zx = jnp.zeros((ZROWS, XP), jnp.float32)
  za = jnp.zeros((ZROWS, AE), jnp.float32)

  aggx = _get_sc_x()(src2d, dst2d, xpad, zx)
  agge = _get_sc_attr()(dst2d, attr8, za)

  batch3 = jnp.pad(batch, (0, NPAD - N)).reshape(NBLK, 1, 1024)
  tdf = target_data.transpose(0, 2, 1).reshape(G, 126)
  w1a = jnp.concatenate(
      [W1[:20], jnp.zeros((XP - 20, 128), jnp.float32)], axis=0)
  w1b = jnp.concatenate(
      [W1[20:25], jnp.zeros((AE - 5, 128), jnp.float32)], axis=0)
  wflat = conv_w.transpose(2, 1, 0).reshape(105, 64)
  return _tc_head(
      aggx, agge, batch3, tdf, w1a, w1b, b1.reshape(1, 128),
      W2, b2.reshape(1, 256), wflat, conv_b.reshape(1, 64),
      cfc_w, cfc_b.reshape(1, 128), out_w[:256], out_w[256:],
      out_b.reshape(1, 1))
